# baseline (speedup 1.0000x reference)
import functools

import jax
import jax.numpy as jnp
import numpy as np
from jax import lax
from jax.experimental import pallas as pl
from jax.experimental.pallas import tpu as pltpu
from jax.experimental.pallas import tpu_sc as plsc

B = 4
C = 384
HW = 224 * 224
RC = B * C
FR_R = B * 3
K = 8
CB = 1792
NSTEPS = HW // CB
NLANE = CB // 128
INV_N = 1.0 / HW
I32_MAX = np.int32(2147483647)
NEG_INF = np.float32(-np.inf)


def _reduce_body(f1_ref, f2_ref, fr_ref, s1_ref, s2_ref, sfr_ref,
                 acc1, acc2, accf):
    step = pl.program_id(0)

    @pl.when(step == 0)
    def _():
        acc1[...] = jnp.zeros_like(acc1)
        acc2[...] = jnp.zeros_like(acc2)
        accf[...] = jnp.zeros_like(accf)

    def accum(src_ref, acc_ref, round_bf16=False):
        x = src_ref[...]
        if round_bf16:
            x = x.astype(jnp.bfloat16).astype(jnp.float32)
        p = x[:, 0:128]
        for i in range(1, NLANE):
            p = p + x[:, i * 128:(i + 1) * 128]
        acc_ref[...] += p

    accum(f1_ref, acc1)
    accum(f2_ref, acc2)
    accum(fr_ref, accf, round_bf16=True)

    @pl.when(step == NSTEPS - 1)
    def _():
        s1_ref[...] = jnp.sum(acc1[...], axis=1, keepdims=True)
        s2_ref[...] = jnp.sum(acc2[...], axis=1, keepdims=True)
        sfr_ref[...] = jnp.sum(accf[...], axis=1, keepdims=True)


_reduce_call = pl.pallas_call(
    _reduce_body,
    grid=(NSTEPS,),
    in_specs=[
        pl.BlockSpec((RC, CB), lambda i: (0, i)),
        pl.BlockSpec((RC, CB), lambda i: (0, i)),
        pl.BlockSpec((FR_R, CB), lambda i: (0, i)),
    ],
    out_specs=[
        pl.BlockSpec((RC, 1), lambda i: (0, 0)),
        pl.BlockSpec((RC, 1), lambda i: (0, 0)),
        pl.BlockSpec((FR_R, 1), lambda i: (0, 0)),
    ],
    out_shape=[
        jax.ShapeDtypeStruct((RC, 1), jnp.float32),
        jax.ShapeDtypeStruct((RC, 1), jnp.float32),
        jax.ShapeDtypeStruct((FR_R, 1), jnp.float32),
    ],
    scratch_shapes=[
        pltpu.VMEM((RC, 128), jnp.float32),
        pltpu.VMEM((RC, 128), jnp.float32),
        pltpu.VMEM((FR_R, 128), jnp.float32),
    ],
    compiler_params=pltpu.CompilerParams(
        dimension_semantics=("arbitrary",),
    ),
)


NCHUNK = C // 16


def _bf16_round(v):
    u = plsc.bitcast(v, jnp.int32)
    lsb = jnp.bitwise_and(jax.lax.shift_right_logical(u, 16), jnp.int32(1))
    t = u + jnp.int32(0x7FFF) + lsb
    t = jnp.bitwise_and(t, jnp.int32(-65536))
    return plsc.bitcast(t, jnp.float32)


def _gate_body(s1_ref, s2_ref, sfr_ref, wt_ref,
                 w1_ref, i1_ref, w2_ref, i2_ref,
                 wt_v, fm_v, x_v, vals_v, wbuf_v, ibuf_v):
    cid = lax.axis_index("c")
    sid = lax.axis_index("s")
    wid = sid * 2 + cid

    lanes = lax.iota(jnp.int32, 16)

    def run_task(b, x_hbm, w_hbm, i_hbm):
        pltpu.sync_copy(wt_ref, wt_v)
        pltpu.sync_copy(sfr_ref, fm_v)
        xoff = pl.multiple_of(b * C, 8)
        pltpu.sync_copy(x_hbm.at[pl.ds(xoff, C)], x_v)

        inv = jnp.float32(INV_N)
        f0 = plsc.load_gather(fm_v, [jnp.broadcast_to(3 * b + 0, (16,))]) * inv
        f1 = plsc.load_gather(fm_v, [jnp.broadcast_to(3 * b + 1, (16,))]) * inv
        f2 = plsc.load_gather(fm_v, [jnp.broadcast_to(3 * b + 2, (16,))]) * inv

        for i in range(NCHUNK):
            w0 = _bf16_round(wt_v[pl.ds(0 * C + i * 16, 16)])
            w1 = _bf16_round(wt_v[pl.ds(1 * C + i * 16, 16)])
            w2 = _bf16_round(wt_v[pl.ds(2 * C + i * 16, 16)])
            frp = f0 * w0 + f1 * w1 + f2 * w2
            x = x_v[pl.ds(i * 16, 16)] * inv
            vals_v[pl.ds(i * 16, 16)] = -jnp.abs(frp - x)

        tv = jnp.broadcast_to(NEG_INF, (16,))
        ti = jnp.zeros((16,), jnp.int32)
        for k in range(K):
            bv = jnp.broadcast_to(NEG_INF, (16,))
            bi = jnp.broadcast_to(I32_MAX, (16,))
            for i in range(NCHUNK):
                v = vals_v[pl.ds(i * 16, 16)]
                idx = lanes + jnp.int32(i * 16)
                take = (v > bv) | ((v == bv) & (idx < bi))
                bv = jnp.where(take, v, bv)
                bi = jnp.where(take, idx, bi)
            m = jnp.max(bv)
            cand = jnp.where(bv == m, bi, I32_MAX)
            mi = jnp.min(cand)
            tv = jnp.where(lanes == k, m, tv)
            ti = jnp.where(lanes == k, mi, ti)
            plsc.store_scatter(vals_v, [jnp.broadcast_to(mi, (16,))],
                               jnp.broadcast_to(NEG_INF, (16,)),
                               mask=lanes == 0)

        mx = jnp.max(tv)
        e = jnp.exp(tv - mx)
        s = jnp.sum(e)
        wbuf_v[...] = e / s
        ibuf_v[...] = ti
        ooff = pl.multiple_of(b * K, 8)
        pltpu.sync_copy(wbuf_v.at[pl.ds(0, K)], w_hbm.at[pl.ds(ooff, K)])
        pltpu.sync_copy(ibuf_v.at[pl.ds(0, K)], i_hbm.at[pl.ds(ooff, K)])

    @pl.when(wid < 4)
    def _():
        run_task(wid, s1_ref, w1_ref, i1_ref)

    @pl.when((wid >= 4) & (wid < 8))
    def _():
        run_task(wid - 4, s2_ref, w2_ref, i2_ref)


@functools.cache
def _gate_kernel():
    mesh = plsc.VectorSubcoreMesh(core_axis_name="c", subcore_axis_name="s")
    return pl.kernel(
        _gate_body,
        out_type=[
            jax.ShapeDtypeStruct((B * K,), jnp.float32),
            jax.ShapeDtypeStruct((B * K,), jnp.int32),
            jax.ShapeDtypeStruct((B * K,), jnp.float32),
            jax.ShapeDtypeStruct((B * K,), jnp.int32),
        ],
        mesh=mesh,
        scratch_types=[
            pltpu.VMEM((3 * C,), jnp.float32),
            pltpu.VMEM((16,), jnp.float32),
            pltpu.VMEM((C,), jnp.float32),
            pltpu.VMEM((C,), jnp.float32),
            pltpu.VMEM((16,), jnp.float32),
            pltpu.VMEM((16,), jnp.int32),
        ],
        compiler_params=pltpu.CompilerParams(needs_layout_passes=False),
    )


def kernel(F1, F2, fr, W):
    f1v = F1.reshape(RC, HW)
    f2v = F2.reshape(RC, HW)
    frv = fr.reshape(FR_R, HW)
    s1, s2, sfr = _reduce_call(f1v, f2v, frv)
    s1f = s1.reshape(RC)
    s2f = s2.reshape(RC)
    sfrf = jnp.pad(sfr.reshape(FR_R), (0, 16 - FR_R))
    wtf = W.T.reshape(3 * C)
    w1, i1, w2, i2 = _gate_kernel()(s1f, s2f, sfrf, wtf)
    return (w1.reshape(B, K), i1.reshape(B, K),
            w2.reshape(B, K), i2.reshape(B, K))


# speedup vs baseline: 1.7184x; 1.7184x over previous
import functools

import jax
import jax.numpy as jnp
import numpy as np
from jax import lax
from jax.experimental import pallas as pl
from jax.experimental.pallas import tpu as pltpu
from jax.experimental.pallas import tpu_sc as plsc

B = 4
C = 384
HW = 224 * 224
RC = B * C
FR_R = B * 3
K = 8
CB = 1792
NSTEPS = HW // CB
NLANE = CB // 128
INV_N = 1.0 / HW
I32_MAX = np.int32(2147483647)
NEG_INF = np.float32(-np.inf)


RB = 48
NROW_STEPS = RC // RB


def _reduce_body(f1_ref, f2_ref, fr_ref, s1_ref, s2_ref, sfr_ref):
    step = pl.program_id(0)
    s1_ref[...] = jnp.sum(f1_ref[...], axis=(1, 2))[:, None]
    s2_ref[...] = jnp.sum(f2_ref[...], axis=(1, 2))[:, None]

    @pl.when(step == 0)
    def _():
        xf = fr_ref[...].astype(jnp.bfloat16).astype(jnp.float32)
        sfr_ref[...] = jnp.sum(xf, axis=(1, 2))[:, None]


_reduce_call = pl.pallas_call(
    _reduce_body,
    grid=(NROW_STEPS,),
    in_specs=[
        pl.BlockSpec((RB, 224, 224), lambda i: (i, 0, 0)),
        pl.BlockSpec((RB, 224, 224), lambda i: (i, 0, 0)),
        pl.BlockSpec((FR_R, 224, 224), lambda i: (0, 0, 0)),
    ],
    out_specs=[
        pl.BlockSpec((RB, 1), lambda i: (i, 0)),
        pl.BlockSpec((RB, 1), lambda i: (i, 0)),
        pl.BlockSpec((FR_R, 1), lambda i: (0, 0)),
    ],
    out_shape=[
        jax.ShapeDtypeStruct((RC, 1), jnp.float32),
        jax.ShapeDtypeStruct((RC, 1), jnp.float32),
        jax.ShapeDtypeStruct((FR_R, 1), jnp.float32),
    ],
    compiler_params=pltpu.CompilerParams(
        dimension_semantics=("arbitrary",),
    ),
)


NCHUNK = C // 16


def _bf16_round(v):
    u = plsc.bitcast(v, jnp.int32)
    lsb = jnp.bitwise_and(jax.lax.shift_right_logical(u, 16), jnp.int32(1))
    t = u + jnp.int32(0x7FFF) + lsb
    t = jnp.bitwise_and(t, jnp.int32(-65536))
    return plsc.bitcast(t, jnp.float32)


def _gate_body(s1_ref, s2_ref, sfr_ref, wt_ref,
                 w1_ref, i1_ref, w2_ref, i2_ref,
                 wt_v, fm_v, x_v, vals_v, wbuf_v, ibuf_v):
    cid = lax.axis_index("c")
    sid = lax.axis_index("s")
    wid = sid * 2 + cid

    lanes = lax.iota(jnp.int32, 16)

    def run_task(b, x_hbm, w_hbm, i_hbm):
        pltpu.sync_copy(wt_ref, wt_v)
        pltpu.sync_copy(sfr_ref, fm_v)
        xoff = pl.multiple_of(b * C, 8)
        pltpu.sync_copy(x_hbm.at[pl.ds(xoff, C)], x_v)

        inv = jnp.float32(INV_N)
        f0 = plsc.load_gather(fm_v, [jnp.broadcast_to(3 * b + 0, (16,))]) * inv
        f1 = plsc.load_gather(fm_v, [jnp.broadcast_to(3 * b + 1, (16,))]) * inv
        f2 = plsc.load_gather(fm_v, [jnp.broadcast_to(3 * b + 2, (16,))]) * inv

        for i in range(NCHUNK):
            w0 = _bf16_round(wt_v[pl.ds(0 * C + i * 16, 16)])
            w1 = _bf16_round(wt_v[pl.ds(1 * C + i * 16, 16)])
            w2 = _bf16_round(wt_v[pl.ds(2 * C + i * 16, 16)])
            frp = f0 * w0 + f1 * w1 + f2 * w2
            x = x_v[pl.ds(i * 16, 16)] * inv
            vals_v[pl.ds(i * 16, 16)] = -jnp.abs(frp - x)

        tv = jnp.broadcast_to(NEG_INF, (16,))
        ti = jnp.zeros((16,), jnp.int32)
        for k in range(K):
            bv = jnp.broadcast_to(NEG_INF, (16,))
            bi = jnp.broadcast_to(I32_MAX, (16,))
            for i in range(NCHUNK):
                v = vals_v[pl.ds(i * 16, 16)]
                idx = lanes + jnp.int32(i * 16)
                take = (v > bv) | ((v == bv) & (idx < bi))
                bv = jnp.where(take, v, bv)
                bi = jnp.where(take, idx, bi)
            m = jnp.max(bv)
            cand = jnp.where(bv == m, bi, I32_MAX)
            mi = jnp.min(cand)
            tv = jnp.where(lanes == k, m, tv)
            ti = jnp.where(lanes == k, mi, ti)
            plsc.store_scatter(vals_v, [jnp.broadcast_to(mi, (16,))],
                               jnp.broadcast_to(NEG_INF, (16,)),
                               mask=lanes == 0)

        mx = jnp.max(tv)
        e = jnp.exp(tv - mx)
        s = jnp.sum(e)
        wbuf_v[...] = e / s
        ibuf_v[...] = ti
        ooff = pl.multiple_of(b * K, 8)
        pltpu.sync_copy(wbuf_v.at[pl.ds(0, K)], w_hbm.at[pl.ds(ooff, K)])
        pltpu.sync_copy(ibuf_v.at[pl.ds(0, K)], i_hbm.at[pl.ds(ooff, K)])

    @pl.when(wid < 4)
    def _():
        run_task(wid, s1_ref, w1_ref, i1_ref)

    @pl.when((wid >= 4) & (wid < 8))
    def _():
        run_task(wid - 4, s2_ref, w2_ref, i2_ref)


@functools.cache
def _gate_kernel():
    mesh = plsc.VectorSubcoreMesh(core_axis_name="c", subcore_axis_name="s")
    return pl.kernel(
        _gate_body,
        out_type=[
            jax.ShapeDtypeStruct((B * K,), jnp.float32),
            jax.ShapeDtypeStruct((B * K,), jnp.int32),
            jax.ShapeDtypeStruct((B * K,), jnp.float32),
            jax.ShapeDtypeStruct((B * K,), jnp.int32),
        ],
        mesh=mesh,
        scratch_types=[
            pltpu.VMEM((3 * C,), jnp.float32),
            pltpu.VMEM((16,), jnp.float32),
            pltpu.VMEM((C,), jnp.float32),
            pltpu.VMEM((C,), jnp.float32),
            pltpu.VMEM((16,), jnp.float32),
            pltpu.VMEM((16,), jnp.int32),
        ],
        compiler_params=pltpu.CompilerParams(needs_layout_passes=False),
    )


def kernel(F1, F2, fr, W):
    f1v = F1.reshape(RC, 224, 224)
    f2v = F2.reshape(RC, 224, 224)
    frv = fr.reshape(FR_R, 224, 224)
    s1, s2, sfr = _reduce_call(f1v, f2v, frv)
    s1f = s1.reshape(RC)
    s2f = s2.reshape(RC)
    sfrf = jnp.pad(sfr.reshape(FR_R), (0, 16 - FR_R))
    wtf = W.T.reshape(3 * C)
    w1, i1, w2, i2 = _gate_kernel()(s1f, s2f, sfrf, wtf)
    return (w1.reshape(B, K), i1.reshape(B, K),
            w2.reshape(B, K), i2.reshape(B, K))


# speedup vs baseline: 5.9098x; 3.4391x over previous
import functools

import jax
import jax.numpy as jnp
import numpy as np
from jax import lax
from jax.experimental import pallas as pl
from jax.experimental.pallas import tpu as pltpu
from jax.experimental.pallas import tpu_sc as plsc

B = 4
C = 384
HW = 224 * 224
RC = B * C
FR_R = B * 3
K = 8
CB = 1792
NSTEPS = HW // CB
NLANE = CB // 128
INV_N = 1.0 / HW
I32_MAX = np.int32(2147483647)
NEG_INF = np.float32(-np.inf)


HB = 28
NH_STEPS = 224 // HB


def _reduce_body(f1_ref, f2_ref, fr_ref, s1_ref, s2_ref, sfr_ref):
    b = pl.program_id(0)
    jh = pl.program_id(1)
    p1 = jnp.sum(f1_ref[...], axis=(0, 1))[None, None, :]
    p2 = jnp.sum(f2_ref[...], axis=(0, 1))[None, None, :]

    @pl.when(jh == 0)
    def _():
        s1_ref[...] = p1
        s2_ref[...] = p2

    @pl.when(jh != 0)
    def _():
        s1_ref[...] += p1
        s2_ref[...] += p2

    @pl.when((b == 0) & (jh == 0))
    def _():
        xf = fr_ref[...].astype(jnp.bfloat16).astype(jnp.float32)
        sfr_ref[...] = jnp.sum(xf, axis=(1, 2))[:, None]


_reduce_call = pl.pallas_call(
    _reduce_body,
    grid=(B, NH_STEPS),
    in_specs=[
        pl.BlockSpec((HB, 224, C), lambda b, j: (b * NH_STEPS + j, 0, 0)),
        pl.BlockSpec((HB, 224, C), lambda b, j: (b * NH_STEPS + j, 0, 0)),
        pl.BlockSpec((FR_R, 224, 224), lambda b, j: (0, 0, 0)),
    ],
    out_specs=[
        pl.BlockSpec((1, 1, C), lambda b, j: (b, 0, 0)),
        pl.BlockSpec((1, 1, C), lambda b, j: (b, 0, 0)),
        pl.BlockSpec((FR_R, 1), lambda b, j: (0, 0)),
    ],
    out_shape=[
        jax.ShapeDtypeStruct((B, 1, C), jnp.float32),
        jax.ShapeDtypeStruct((B, 1, C), jnp.float32),
        jax.ShapeDtypeStruct((FR_R, 1), jnp.float32),
    ],
    compiler_params=pltpu.CompilerParams(
        dimension_semantics=("arbitrary", "arbitrary"),
    ),
)


NCHUNK = C // 16


def _bf16_round(v):
    u = plsc.bitcast(v, jnp.int32)
    lsb = jnp.bitwise_and(jax.lax.shift_right_logical(u, 16), jnp.int32(1))
    t = u + jnp.int32(0x7FFF) + lsb
    t = jnp.bitwise_and(t, jnp.int32(-65536))
    return plsc.bitcast(t, jnp.float32)


def _gate_body(s1_ref, s2_ref, sfr_ref, wt_ref,
                 w1_ref, i1_ref, w2_ref, i2_ref,
                 wt_v, fm_v, x_v, vals_v, wbuf_v, ibuf_v):
    cid = lax.axis_index("c")
    sid = lax.axis_index("s")
    wid = sid * 2 + cid

    lanes = lax.iota(jnp.int32, 16)

    def run_task(b, x_hbm, w_hbm, i_hbm):
        pltpu.sync_copy(wt_ref, wt_v)
        pltpu.sync_copy(sfr_ref, fm_v)
        xoff = pl.multiple_of(b * C, 8)
        pltpu.sync_copy(x_hbm.at[pl.ds(xoff, C)], x_v)

        inv = jnp.float32(INV_N)
        f0 = plsc.load_gather(fm_v, [jnp.broadcast_to(3 * b + 0, (16,))]) * inv
        f1 = plsc.load_gather(fm_v, [jnp.broadcast_to(3 * b + 1, (16,))]) * inv
        f2 = plsc.load_gather(fm_v, [jnp.broadcast_to(3 * b + 2, (16,))]) * inv

        for i in range(NCHUNK):
            w0 = _bf16_round(wt_v[pl.ds(0 * C + i * 16, 16)])
            w1 = _bf16_round(wt_v[pl.ds(1 * C + i * 16, 16)])
            w2 = _bf16_round(wt_v[pl.ds(2 * C + i * 16, 16)])
            frp = f0 * w0 + f1 * w1 + f2 * w2
            x = x_v[pl.ds(i * 16, 16)] * inv
            vals_v[pl.ds(i * 16, 16)] = -jnp.abs(frp - x)

        tv = jnp.broadcast_to(NEG_INF, (16,))
        ti = jnp.zeros((16,), jnp.int32)
        for k in range(K):
            bv = jnp.broadcast_to(NEG_INF, (16,))
            bi = jnp.broadcast_to(I32_MAX, (16,))
            for i in range(NCHUNK):
                v = vals_v[pl.ds(i * 16, 16)]
                idx = lanes + jnp.int32(i * 16)
                take = (v > bv) | ((v == bv) & (idx < bi))
                bv = jnp.where(take, v, bv)
                bi = jnp.where(take, idx, bi)
            m = jnp.max(bv)
            cand = jnp.where(bv == m, bi, I32_MAX)
            mi = jnp.min(cand)
            tv = jnp.where(lanes == k, m, tv)
            ti = jnp.where(lanes == k, mi, ti)
            plsc.store_scatter(vals_v, [jnp.broadcast_to(mi, (16,))],
                               jnp.broadcast_to(NEG_INF, (16,)),
                               mask=lanes == 0)

        mx = jnp.max(tv)
        e = jnp.exp(tv - mx)
        s = jnp.sum(e)
        wbuf_v[...] = e / s
        ibuf_v[...] = ti
        ooff = pl.multiple_of(b * K, 8)
        pltpu.sync_copy(wbuf_v.at[pl.ds(0, K)], w_hbm.at[pl.ds(ooff, K)])
        pltpu.sync_copy(ibuf_v.at[pl.ds(0, K)], i_hbm.at[pl.ds(ooff, K)])

    @pl.when(wid < 4)
    def _():
        run_task(wid, s1_ref, w1_ref, i1_ref)

    @pl.when((wid >= 4) & (wid < 8))
    def _():
        run_task(wid - 4, s2_ref, w2_ref, i2_ref)


@functools.cache
def _gate_kernel():
    mesh = plsc.VectorSubcoreMesh(core_axis_name="c", subcore_axis_name="s")
    return pl.kernel(
        _gate_body,
        out_type=[
            jax.ShapeDtypeStruct((B * K,), jnp.float32),
            jax.ShapeDtypeStruct((B * K,), jnp.int32),
            jax.ShapeDtypeStruct((B * K,), jnp.float32),
            jax.ShapeDtypeStruct((B * K,), jnp.int32),
        ],
        mesh=mesh,
        scratch_types=[
            pltpu.VMEM((3 * C,), jnp.float32),
            pltpu.VMEM((16,), jnp.float32),
            pltpu.VMEM((C,), jnp.float32),
            pltpu.VMEM((C,), jnp.float32),
            pltpu.VMEM((16,), jnp.float32),
            pltpu.VMEM((16,), jnp.int32),
        ],
        compiler_params=pltpu.CompilerParams(needs_layout_passes=False),
    )


def kernel(F1, F2, fr, W):
    f1v = jnp.transpose(F1, (0, 2, 3, 1)).reshape(B * 224, 224, C)
    f2v = jnp.transpose(F2, (0, 2, 3, 1)).reshape(B * 224, 224, C)
    frv = fr.reshape(FR_R, 224, 224)
    s1, s2, sfr = _reduce_call(f1v, f2v, frv)
    s1f = s1.reshape(RC)
    s2f = s2.reshape(RC)
    sfrf = jnp.pad(sfr.reshape(FR_R), (0, 16 - FR_R))
    wtf = W.T.reshape(3 * C)
    w1, i1, w2, i2 = _gate_kernel()(s1f, s2f, sfrf, wtf)
    return (w1.reshape(B, K), i1.reshape(B, K),
            w2.reshape(B, K), i2.reshape(B, K))


# speedup vs baseline: 5.9219x; 1.0020x over previous
import functools

import jax
import jax.numpy as jnp
import numpy as np
from jax import lax
from jax.experimental import pallas as pl
from jax.experimental.pallas import tpu as pltpu
from jax.experimental.pallas import tpu_sc as plsc

B = 4
C = 384
HW = 224 * 224
RC = B * C
FR_R = B * 3
K = 8
CB = 1792
NSTEPS = HW // CB
NLANE = CB // 128
INV_N = 1.0 / HW
I32_MAX = np.int32(2147483647)
NEG_INF = np.float32(-np.inf)


HB = 32
NH_STEPS = 224 // HB


def _reduce_body(f1_ref, f2_ref, fr_ref, s1_ref, s2_ref, sfr_ref):
    b = pl.program_id(0)
    jh = pl.program_id(1)
    p1 = jnp.sum(f1_ref[...], axis=(0, 1))[None, None, :]
    p2 = jnp.sum(f2_ref[...], axis=(0, 1))[None, None, :]

    @pl.when(jh == 0)
    def _():
        s1_ref[...] = p1
        s2_ref[...] = p2

    @pl.when(jh != 0)
    def _():
        s1_ref[...] += p1
        s2_ref[...] += p2

    @pl.when((b == 0) & (jh == 0))
    def _():
        xf = fr_ref[...].astype(jnp.bfloat16).astype(jnp.float32)
        sfr_ref[...] = jnp.sum(xf, axis=(1, 2))[:, None]


_reduce_call = pl.pallas_call(
    _reduce_body,
    grid=(B, NH_STEPS),
    in_specs=[
        pl.BlockSpec((HB, 224, C), lambda b, j: (b * NH_STEPS + j, 0, 0)),
        pl.BlockSpec((HB, 224, C), lambda b, j: (b * NH_STEPS + j, 0, 0)),
        pl.BlockSpec((FR_R, 224, 224), lambda b, j: (0, 0, 0)),
    ],
    out_specs=[
        pl.BlockSpec((1, 1, C), lambda b, j: (b, 0, 0)),
        pl.BlockSpec((1, 1, C), lambda b, j: (b, 0, 0)),
        pl.BlockSpec((FR_R, 1), lambda b, j: (0, 0)),
    ],
    out_shape=[
        jax.ShapeDtypeStruct((B, 1, C), jnp.float32),
        jax.ShapeDtypeStruct((B, 1, C), jnp.float32),
        jax.ShapeDtypeStruct((FR_R, 1), jnp.float32),
    ],
    compiler_params=pltpu.CompilerParams(
        dimension_semantics=("arbitrary", "arbitrary"),
        vmem_limit_bytes=60 * 1024 * 1024,
    ),
)


NCHUNK = C // 16


def _bf16_round(v):
    u = plsc.bitcast(v, jnp.int32)
    lsb = jnp.bitwise_and(jax.lax.shift_right_logical(u, 16), jnp.int32(1))
    t = u + jnp.int32(0x7FFF) + lsb
    t = jnp.bitwise_and(t, jnp.int32(-65536))
    return plsc.bitcast(t, jnp.float32)


def _gate_body(s1_ref, s2_ref, sfr_ref, wt_ref,
                 w1_ref, i1_ref, w2_ref, i2_ref,
                 wt_v, fm_v, x_v, vals_v, wbuf_v, ibuf_v):
    cid = lax.axis_index("c")
    sid = lax.axis_index("s")
    wid = sid * 2 + cid

    lanes = lax.iota(jnp.int32, 16)

    def run_task(b, x_hbm, w_hbm, i_hbm):
        pltpu.sync_copy(wt_ref, wt_v)
        pltpu.sync_copy(sfr_ref, fm_v)
        xoff = pl.multiple_of(b * C, 8)
        pltpu.sync_copy(x_hbm.at[pl.ds(xoff, C)], x_v)

        inv = jnp.float32(INV_N)
        f0 = plsc.load_gather(fm_v, [jnp.broadcast_to(3 * b + 0, (16,))]) * inv
        f1 = plsc.load_gather(fm_v, [jnp.broadcast_to(3 * b + 1, (16,))]) * inv
        f2 = plsc.load_gather(fm_v, [jnp.broadcast_to(3 * b + 2, (16,))]) * inv

        for i in range(NCHUNK):
            w0 = _bf16_round(wt_v[pl.ds(0 * C + i * 16, 16)])
            w1 = _bf16_round(wt_v[pl.ds(1 * C + i * 16, 16)])
            w2 = _bf16_round(wt_v[pl.ds(2 * C + i * 16, 16)])
            frp = f0 * w0 + f1 * w1 + f2 * w2
            x = x_v[pl.ds(i * 16, 16)] * inv
            vals_v[pl.ds(i * 16, 16)] = -jnp.abs(frp - x)

        tv = jnp.broadcast_to(NEG_INF, (16,))
        ti = jnp.zeros((16,), jnp.int32)
        for k in range(K):
            bv = jnp.broadcast_to(NEG_INF, (16,))
            bi = jnp.broadcast_to(I32_MAX, (16,))
            for i in range(NCHUNK):
                v = vals_v[pl.ds(i * 16, 16)]
                idx = lanes + jnp.int32(i * 16)
                take = (v > bv) | ((v == bv) & (idx < bi))
                bv = jnp.where(take, v, bv)
                bi = jnp.where(take, idx, bi)
            m = jnp.max(bv)
            cand = jnp.where(bv == m, bi, I32_MAX)
            mi = jnp.min(cand)
            tv = jnp.where(lanes == k, m, tv)
            ti = jnp.where(lanes == k, mi, ti)
            plsc.store_scatter(vals_v, [jnp.broadcast_to(mi, (16,))],
                               jnp.broadcast_to(NEG_INF, (16,)),
                               mask=lanes == 0)

        mx = jnp.max(tv)
        e = jnp.exp(tv - mx)
        s = jnp.sum(e)
        wbuf_v[...] = e / s
        ibuf_v[...] = ti
        ooff = pl.multiple_of(b * K, 8)
        pltpu.sync_copy(wbuf_v.at[pl.ds(0, K)], w_hbm.at[pl.ds(ooff, K)])
        pltpu.sync_copy(ibuf_v.at[pl.ds(0, K)], i_hbm.at[pl.ds(ooff, K)])

    @pl.when(wid < 4)
    def _():
        run_task(wid, s1_ref, w1_ref, i1_ref)

    @pl.when((wid >= 4) & (wid < 8))
    def _():
        run_task(wid - 4, s2_ref, w2_ref, i2_ref)


@functools.cache
def _gate_kernel():
    mesh = plsc.VectorSubcoreMesh(core_axis_name="c", subcore_axis_name="s")
    return pl.kernel(
        _gate_body,
        out_type=[
            jax.ShapeDtypeStruct((B * K,), jnp.float32),
            jax.ShapeDtypeStruct((B * K,), jnp.int32),
            jax.ShapeDtypeStruct((B * K,), jnp.float32),
            jax.ShapeDtypeStruct((B * K,), jnp.int32),
        ],
        mesh=mesh,
        scratch_types=[
            pltpu.VMEM((3 * C,), jnp.float32),
            pltpu.VMEM((16,), jnp.float32),
            pltpu.VMEM((C,), jnp.float32),
            pltpu.VMEM((C,), jnp.float32),
            pltpu.VMEM((16,), jnp.float32),
            pltpu.VMEM((16,), jnp.int32),
        ],
        compiler_params=pltpu.CompilerParams(needs_layout_passes=False),
    )


def kernel(F1, F2, fr, W):
    f1v = jnp.transpose(F1, (0, 2, 3, 1)).reshape(B * 224, 224, C)
    f2v = jnp.transpose(F2, (0, 2, 3, 1)).reshape(B * 224, 224, C)
    frv = fr.reshape(FR_R, 224, 224)
    s1, s2, sfr = _reduce_call(f1v, f2v, frv)
    s1f = s1.reshape(RC)
    s2f = s2.reshape(RC)
    sfrf = jnp.pad(sfr.reshape(FR_R), (0, 16 - FR_R))
    wtf = W.T.reshape(3 * C)
    w1, i1, w2, i2 = _gate_kernel()(s1f, s2f, sfrf, wtf)
    return (w1.reshape(B, K), i1.reshape(B, K),
            w2.reshape(B, K), i2.reshape(B, K))


# speedup vs baseline: 5.9383x; 1.0028x over previous
import functools

import jax
import jax.numpy as jnp
import numpy as np
from jax import lax
from jax.experimental import pallas as pl
from jax.experimental.pallas import tpu as pltpu
from jax.experimental.pallas import tpu_sc as plsc

B = 4
C = 384
HW = 224 * 224
RC = B * C
FR_R = B * 3
K = 8
CB = 1792
NSTEPS = HW // CB
NLANE = CB // 128
INV_N = 1.0 / HW
I32_MAX = np.int32(2147483647)
NEG_INF = np.float32(-np.inf)


HB = 16
NH_STEPS = 224 // HB


def _reduce_body(f1_ref, f2_ref, fr_ref, s1_ref, s2_ref, sfr_ref):
    b = pl.program_id(0)
    jh = pl.program_id(1)
    p1 = jnp.sum(f1_ref[...], axis=(0, 1))[None, None, :]
    p2 = jnp.sum(f2_ref[...], axis=(0, 1))[None, None, :]

    @pl.when(jh == 0)
    def _():
        s1_ref[...] = p1
        s2_ref[...] = p2

    @pl.when(jh != 0)
    def _():
        s1_ref[...] += p1
        s2_ref[...] += p2

    @pl.when((b == 0) & (jh == 0))
    def _():
        xf = fr_ref[...].astype(jnp.bfloat16).astype(jnp.float32)
        sfr_ref[...] = jnp.sum(xf, axis=(1, 2))[:, None]


_reduce_call = pl.pallas_call(
    _reduce_body,
    grid=(B, NH_STEPS),
    in_specs=[
        pl.BlockSpec((HB, 224, C), lambda b, j: (b * NH_STEPS + j, 0, 0)),
        pl.BlockSpec((HB, 224, C), lambda b, j: (b * NH_STEPS + j, 0, 0)),
        pl.BlockSpec((FR_R, 224, 224), lambda b, j: (0, 0, 0)),
    ],
    out_specs=[
        pl.BlockSpec((1, 1, C), lambda b, j: (b, 0, 0)),
        pl.BlockSpec((1, 1, C), lambda b, j: (b, 0, 0)),
        pl.BlockSpec((FR_R, 1), lambda b, j: (0, 0)),
    ],
    out_shape=[
        jax.ShapeDtypeStruct((B, 1, C), jnp.float32),
        jax.ShapeDtypeStruct((B, 1, C), jnp.float32),
        jax.ShapeDtypeStruct((FR_R, 1), jnp.float32),
    ],
    compiler_params=pltpu.CompilerParams(
        dimension_semantics=("arbitrary", "arbitrary"),
        vmem_limit_bytes=60 * 1024 * 1024,
    ),
)


NCHUNK = C // 16


def _bf16_round(v):
    u = plsc.bitcast(v, jnp.int32)
    lsb = jnp.bitwise_and(jax.lax.shift_right_logical(u, 16), jnp.int32(1))
    t = u + jnp.int32(0x7FFF) + lsb
    t = jnp.bitwise_and(t, jnp.int32(-65536))
    return plsc.bitcast(t, jnp.float32)


def _gate_body(s1_ref, s2_ref, sfr_ref, wt_ref,
                 w1_ref, i1_ref, w2_ref, i2_ref,
                 wt_v, fm_v, x_v, vals_v, wbuf_v, ibuf_v):
    cid = lax.axis_index("c")
    sid = lax.axis_index("s")
    wid = sid * 2 + cid

    lanes = lax.iota(jnp.int32, 16)

    def run_task(b, x_hbm, w_hbm, i_hbm):
        pltpu.sync_copy(wt_ref, wt_v)
        pltpu.sync_copy(sfr_ref, fm_v)
        xoff = pl.multiple_of(b * C, 8)
        pltpu.sync_copy(x_hbm.at[pl.ds(xoff, C)], x_v)

        inv = jnp.float32(INV_N)
        f0 = plsc.load_gather(fm_v, [jnp.broadcast_to(3 * b + 0, (16,))]) * inv
        f1 = plsc.load_gather(fm_v, [jnp.broadcast_to(3 * b + 1, (16,))]) * inv
        f2 = plsc.load_gather(fm_v, [jnp.broadcast_to(3 * b + 2, (16,))]) * inv

        for i in range(NCHUNK):
            w0 = _bf16_round(wt_v[pl.ds(0 * C + i * 16, 16)])
            w1 = _bf16_round(wt_v[pl.ds(1 * C + i * 16, 16)])
            w2 = _bf16_round(wt_v[pl.ds(2 * C + i * 16, 16)])
            frp = f0 * w0 + f1 * w1 + f2 * w2
            x = x_v[pl.ds(i * 16, 16)] * inv
            vals_v[pl.ds(i * 16, 16)] = -jnp.abs(frp - x)

        tv = jnp.broadcast_to(NEG_INF, (16,))
        ti = jnp.zeros((16,), jnp.int32)
        for k in range(K):
            bv = jnp.broadcast_to(NEG_INF, (16,))
            bi = jnp.broadcast_to(I32_MAX, (16,))
            for i in range(NCHUNK):
                v = vals_v[pl.ds(i * 16, 16)]
                idx = lanes + jnp.int32(i * 16)
                take = (v > bv) | ((v == bv) & (idx < bi))
                bv = jnp.where(take, v, bv)
                bi = jnp.where(take, idx, bi)
            m = jnp.max(bv)
            cand = jnp.where(bv == m, bi, I32_MAX)
            mi = jnp.min(cand)
            tv = jnp.where(lanes == k, m, tv)
            ti = jnp.where(lanes == k, mi, ti)
            plsc.store_scatter(vals_v, [jnp.broadcast_to(mi, (16,))],
                               jnp.broadcast_to(NEG_INF, (16,)),
                               mask=lanes == 0)

        mx = jnp.max(tv)
        e = jnp.exp(tv - mx)
        s = jnp.sum(e)
        wbuf_v[...] = e / s
        ibuf_v[...] = ti
        ooff = pl.multiple_of(b * K, 8)
        pltpu.sync_copy(wbuf_v.at[pl.ds(0, K)], w_hbm.at[pl.ds(ooff, K)])
        pltpu.sync_copy(ibuf_v.at[pl.ds(0, K)], i_hbm.at[pl.ds(ooff, K)])

    @pl.when(wid < 4)
    def _():
        run_task(wid, s1_ref, w1_ref, i1_ref)

    @pl.when((wid >= 4) & (wid < 8))
    def _():
        run_task(wid - 4, s2_ref, w2_ref, i2_ref)


@functools.cache
def _gate_kernel():
    mesh = plsc.VectorSubcoreMesh(core_axis_name="c", subcore_axis_name="s")
    return pl.kernel(
        _gate_body,
        out_type=[
            jax.ShapeDtypeStruct((B * K,), jnp.float32),
            jax.ShapeDtypeStruct((B * K,), jnp.int32),
            jax.ShapeDtypeStruct((B * K,), jnp.float32),
            jax.ShapeDtypeStruct((B * K,), jnp.int32),
        ],
        mesh=mesh,
        scratch_types=[
            pltpu.VMEM((3 * C,), jnp.float32),
            pltpu.VMEM((16,), jnp.float32),
            pltpu.VMEM((C,), jnp.float32),
            pltpu.VMEM((C,), jnp.float32),
            pltpu.VMEM((16,), jnp.float32),
            pltpu.VMEM((16,), jnp.int32),
        ],
        compiler_params=pltpu.CompilerParams(needs_layout_passes=False),
    )


def kernel(F1, F2, fr, W):
    f1v = jnp.transpose(F1, (0, 2, 3, 1)).reshape(B * 224, 224, C)
    f2v = jnp.transpose(F2, (0, 2, 3, 1)).reshape(B * 224, 224, C)
    frv = fr.reshape(FR_R, 224, 224)
    s1, s2, sfr = _reduce_call(f1v, f2v, frv)
    s1f = s1.reshape(RC)
    s2f = s2.reshape(RC)
    sfrf = jnp.pad(sfr.reshape(FR_R), (0, 16 - FR_R))
    wtf = W.T.reshape(3 * C)
    w1, i1, w2, i2 = _gate_kernel()(s1f, s2f, sfrf, wtf)
    return (w1.reshape(B, K), i1.reshape(B, K),
            w2.reshape(B, K), i2.reshape(B, K))


# speedup vs baseline: 6.0230x; 1.0143x over previous
import functools

import jax
import jax.numpy as jnp
import numpy as np
from jax import lax
from jax.experimental import pallas as pl
from jax.experimental.pallas import tpu as pltpu
from jax.experimental.pallas import tpu_sc as plsc

B = 4
C = 384
HW = 224 * 224
RC = B * C
FR_R = B * 3
K = 8
CB = 1792
NSTEPS = HW // CB
NLANE = CB // 128
INV_N = 1.0 / HW
I32_MAX = np.int32(2147483647)
NEG_INF = np.float32(-np.inf)


HB = 16
NH_STEPS = 224 // HB


def _reduce_body(f1_ref, f2_ref, fr_ref, s1_ref, s2_ref, sfr_ref):
    b = pl.program_id(0)
    jh = pl.program_id(1)
    p1 = jnp.sum(f1_ref[...], axis=(0, 1))[None, None, :]
    p2 = jnp.sum(f2_ref[...], axis=(0, 1))[None, None, :]

    @pl.when(jh == 0)
    def _():
        s1_ref[...] = p1
        s2_ref[...] = p2

    @pl.when(jh != 0)
    def _():
        s1_ref[...] += p1
        s2_ref[...] += p2

    @pl.when((b == 0) & (jh == 0))
    def _():
        xf = fr_ref[...].astype(jnp.bfloat16).astype(jnp.float32)
        sfr_ref[...] = jnp.sum(xf, axis=(1, 2))[:, None]


_reduce_call = pl.pallas_call(
    _reduce_body,
    grid=(B, NH_STEPS),
    in_specs=[
        pl.BlockSpec((HB, 224, C), lambda b, j: (b * NH_STEPS + j, 0, 0)),
        pl.BlockSpec((HB, 224, C), lambda b, j: (b * NH_STEPS + j, 0, 0)),
        pl.BlockSpec((FR_R, 224, 224), lambda b, j: (0, 0, 0)),
    ],
    out_specs=[
        pl.BlockSpec((1, 1, C), lambda b, j: (b, 0, 0)),
        pl.BlockSpec((1, 1, C), lambda b, j: (b, 0, 0)),
        pl.BlockSpec((FR_R, 1), lambda b, j: (0, 0)),
    ],
    out_shape=[
        jax.ShapeDtypeStruct((B, 1, C), jnp.float32),
        jax.ShapeDtypeStruct((B, 1, C), jnp.float32),
        jax.ShapeDtypeStruct((FR_R, 1), jnp.float32),
    ],
    compiler_params=pltpu.CompilerParams(
        dimension_semantics=("arbitrary", "arbitrary"),
        vmem_limit_bytes=60 * 1024 * 1024,
    ),
)


NCHUNK = C // 16


def _bf16_round(v):
    u = plsc.bitcast(v, jnp.int32)
    lsb = jnp.bitwise_and(jax.lax.shift_right_logical(u, 16), jnp.int32(1))
    t = u + jnp.int32(0x7FFF) + lsb
    t = jnp.bitwise_and(t, jnp.int32(-65536))
    return plsc.bitcast(t, jnp.float32)


def _gate_body(s12_ref, sfr_ref, wt_ref, wall_ref, iall_ref,
               wt_v, fm_v, x_v, vals_v, wbuf_v, ibuf_v,
               sem0, sem1, sem2):
    cid = lax.axis_index("c")
    sid = lax.axis_index("s")
    wid = sid * 2 + cid

    lanes = lax.iota(jnp.int32, 16)

    @pl.when(wid < 2 * B)
    def _():
        task = wid
        b = task - (task // B) * B
        cp0 = pltpu.async_copy(wt_ref, wt_v, sem0)
        cp1 = pltpu.async_copy(sfr_ref, fm_v, sem1)
        xoff = pl.multiple_of(task * C, 8)
        cp2 = pltpu.async_copy(s12_ref.at[pl.ds(xoff, C)], x_v, sem2)
        cp0.wait()
        cp1.wait()
        cp2.wait()

        inv = jnp.float32(INV_N)
        f0 = plsc.load_gather(fm_v, [jnp.broadcast_to(3 * b + 0, (16,))]) * inv
        f1 = plsc.load_gather(fm_v, [jnp.broadcast_to(3 * b + 1, (16,))]) * inv
        f2 = plsc.load_gather(fm_v, [jnp.broadcast_to(3 * b + 2, (16,))]) * inv

        for i in range(NCHUNK):
            w0 = _bf16_round(wt_v[pl.ds(0 * C + i * 16, 16)])
            w1 = _bf16_round(wt_v[pl.ds(1 * C + i * 16, 16)])
            w2 = _bf16_round(wt_v[pl.ds(2 * C + i * 16, 16)])
            frp = f0 * w0 + f1 * w1 + f2 * w2
            x = x_v[pl.ds(i * 16, 16)] * inv
            vals_v[pl.ds(i * 16, 16)] = -jnp.abs(frp - x)

        tv = jnp.broadcast_to(NEG_INF, (16,))
        ti = jnp.zeros((16,), jnp.int32)
        for k in range(K):
            bv = jnp.broadcast_to(NEG_INF, (16,))
            bi = jnp.broadcast_to(I32_MAX, (16,))
            for i in range(NCHUNK):
                v = vals_v[pl.ds(i * 16, 16)]
                idx = lanes + jnp.int32(i * 16)
                take = (v > bv) | ((v == bv) & (idx < bi))
                bv = jnp.where(take, v, bv)
                bi = jnp.where(take, idx, bi)
            m = jnp.max(bv)
            cand = jnp.where(bv == m, bi, I32_MAX)
            mi = jnp.min(cand)
            tv = jnp.where(lanes == k, m, tv)
            ti = jnp.where(lanes == k, mi, ti)
            plsc.store_scatter(vals_v, [jnp.broadcast_to(mi, (16,))],
                               jnp.broadcast_to(NEG_INF, (16,)),
                               mask=lanes == 0)

        mx = jnp.max(tv)
        e = jnp.exp(tv - mx)
        s = jnp.sum(e)
        wbuf_v[...] = e / s
        ibuf_v[...] = ti
        ooff = pl.multiple_of(task * K, 8)
        pltpu.sync_copy(wbuf_v.at[pl.ds(0, K)], wall_ref.at[pl.ds(ooff, K)])
        pltpu.sync_copy(ibuf_v.at[pl.ds(0, K)], iall_ref.at[pl.ds(ooff, K)])


@functools.cache
def _gate_kernel():
    mesh = plsc.VectorSubcoreMesh(core_axis_name="c", subcore_axis_name="s")
    return pl.kernel(
        _gate_body,
        out_type=[
            jax.ShapeDtypeStruct((2 * B * K,), jnp.float32),
            jax.ShapeDtypeStruct((2 * B * K,), jnp.int32),
        ],
        mesh=mesh,
        scratch_types=[
            pltpu.VMEM((3 * C,), jnp.float32),
            pltpu.VMEM((16,), jnp.float32),
            pltpu.VMEM((C,), jnp.float32),
            pltpu.VMEM((C,), jnp.float32),
            pltpu.VMEM((16,), jnp.float32),
            pltpu.VMEM((16,), jnp.int32),
            pltpu.SemaphoreType.DMA,
            pltpu.SemaphoreType.DMA,
            pltpu.SemaphoreType.DMA,
        ],
        compiler_params=pltpu.CompilerParams(needs_layout_passes=False),
    )


def kernel(F1, F2, fr, W):
    f1v = jnp.transpose(F1, (0, 2, 3, 1)).reshape(B * 224, 224, C)
    f2v = jnp.transpose(F2, (0, 2, 3, 1)).reshape(B * 224, 224, C)
    frv = fr.reshape(FR_R, 224, 224)
    s1, s2, sfr = _reduce_call(f1v, f2v, frv)
    s12f = jnp.concatenate([s1.reshape(RC), s2.reshape(RC)])
    sfrf = jnp.pad(sfr.reshape(FR_R), (0, 16 - FR_R))
    wtf = W.T.reshape(3 * C)
    wall, iall = _gate_kernel()(s12f, sfrf, wtf)
    wall = wall.reshape(2, B, K)
    iall = iall.reshape(2, B, K)
    return (wall[0], iall[0], wall[1], iall[1])
